# baseline (speedup 1.0000x reference)
import functools

import jax
import jax.numpy as jnp
from jax.experimental import pallas as pl
from jax.experimental.pallas import tpu as pltpu

N = 10000
BR = 128


def _mv_kernel(params_ref, xT_ref, adj_ref, out_ref):
    p = params_ref
    v = (p[0, 0] * xT_ref[0:1, :]
         + p[0, 1] * xT_ref[1:2, :]
         + p[0, 2] * xT_ref[2:3, :]
         + p[0, 3] * xT_ref[3:4, :]
         + p[0, 4])
    acc = jnp.sum(adj_ref[:, :] * v, axis=1, keepdims=True)
    out_ref[:, :] = acc + p[0, 5]


@jax.jit
def kernel(x, adj, W_enc, b_enc, W_dec, b_dec):
    w = (W_dec @ W_enc).reshape(4)
    c = (b_enc @ W_dec.T).reshape(())
    params = jnp.concatenate(
        [w, c[None], b_dec.reshape(1)]).reshape(1, 6).astype(jnp.float32)
    xT = x.T

    grid = (pl.cdiv(N, BR),)
    out = pl.pallas_call(
        _mv_kernel,
        grid=grid,
        in_specs=[
            pl.BlockSpec(memory_space=pltpu.SMEM),
            pl.BlockSpec((4, N), lambda i: (0, 0)),
            pl.BlockSpec((BR, N), lambda i: (i, 0)),
        ],
        out_specs=pl.BlockSpec((BR, 1), lambda i: (i, 0)),
        out_shape=jax.ShapeDtypeStruct((N, 1), jnp.float32),
    )(params, xT, adj)
    return out
